# Initial kernel scaffold; baseline (speedup 1.0000x reference)
import jax
import jax.numpy as jnp
from jax.experimental import pallas as pl


def kernel(x, tok_weight, pos_weight):
    raise NotImplementedError("write your pallas kernel here")


# speedup vs baseline: 3.2925x; 3.2925x over previous
import functools

import jax
import jax.numpy as jnp
from jax import lax
from jax.experimental import pallas as pl
from jax.experimental.pallas import tpu as pltpu
from jax.experimental.pallas import tpu_sc as plsc

NUM_COLORS = 10
D_MODEL = 128
MAX_LEN = 900
B = 1024
L = 900

TOKENS = B * L
NUM_WORKERS = 32
PER_W = TOKENS // NUM_WORKERS
CHUNK = 128
NCHUNK = PER_W // CHUNK
LANES = 16


def _comb_body(tok_ref, pos_ref, out_ref):
    out_ref[...] = pos_ref[...][:, None, :] + tok_ref[...][None, :, :]


def _build_comb(tok, pos):
    return pl.pallas_call(
        _comb_body,
        out_shape=jax.ShapeDtypeStruct((MAX_LEN, NUM_COLORS, D_MODEL),
                                       jnp.float32),
    )(tok, pos)


_sc_mesh = plsc.VectorSubcoreMesh(core_axis_name="c", subcore_axis_name="s")


@functools.partial(
    pl.kernel,
    out_type=jax.ShapeDtypeStruct((TOKENS, D_MODEL), jnp.float32),
    mesh=_sc_mesh,
    scratch_types=[
        pltpu.VMEM((PER_W,), jnp.int32),
        pltpu.VMEM((PER_W,), jnp.int32),
        pltpu.VMEM((CHUNK, D_MODEL), jnp.float32),
        pltpu.SemaphoreType.DMA,
    ],
)
def _sc_gather(comb_hbm, x_hbm, out_hbm, xv, idxv, rowsv, sem):
    wid = lax.axis_index("s") * 2 + lax.axis_index("c")
    base = wid * PER_W

    pltpu.sync_copy(x_hbm.at[pl.ds(base, PER_W)], xv)

    def idx_body(k, carry):
        o = k * LANES
        p = o + lax.iota(jnp.int32, LANES)
        l = lax.rem(p, jnp.int32(L))
        idxv[pl.ds(o, LANES)] = l * NUM_COLORS + xv[pl.ds(o, LANES)]
        return carry

    lax.fori_loop(0, PER_W // LANES, idx_body, 0)

    def chunk_body(i, carry):
        pltpu.async_copy(
            comb_hbm.at[idxv.at[pl.ds(i * CHUNK, CHUNK)]], rowsv, sem
        ).wait()
        pltpu.sync_copy(rowsv, out_hbm.at[pl.ds(base + i * CHUNK, CHUNK)])
        return carry

    lax.fori_loop(0, NCHUNK, chunk_body, 0)


def kernel(x, tok_weight, pos_weight):
    comb = _build_comb(tok_weight, pos_weight)
    comb_flat = comb.reshape(MAX_LEN * NUM_COLORS, D_MODEL)
    x_flat = x.reshape(-1).astype(jnp.int32)
    out = _sc_gather(comb_flat, x_flat)
    return out.reshape(B, L, D_MODEL)


# speedup vs baseline: 3.5320x; 1.0728x over previous
import functools

import jax
import jax.numpy as jnp
from jax import lax
from jax.experimental import pallas as pl
from jax.experimental.pallas import tpu as pltpu
from jax.experimental.pallas import tpu_sc as plsc

NUM_COLORS = 10
D_MODEL = 128
MAX_LEN = 900
B = 1024
L = 900

TOKENS = B * L
NUM_WORKERS = 32
PER_W = TOKENS // NUM_WORKERS
CHUNK = 120
NCHUNK = PER_W // CHUNK
LANES = 16


def _comb_body(tok_ref, pos_ref, out_ref):
    out_ref[...] = pos_ref[...][:, None, :] + tok_ref[...][None, :, :]


def _build_comb(tok, pos):
    return pl.pallas_call(
        _comb_body,
        out_shape=jax.ShapeDtypeStruct((MAX_LEN, NUM_COLORS, D_MODEL),
                                       jnp.float32),
    )(tok, pos)


_sc_mesh = plsc.VectorSubcoreMesh(core_axis_name="c", subcore_axis_name="s")


@functools.partial(
    pl.kernel,
    out_type=jax.ShapeDtypeStruct((TOKENS, D_MODEL), jnp.float32),
    mesh=_sc_mesh,
    scratch_types=[
        pltpu.VMEM((PER_W,), jnp.int32),
        pltpu.VMEM((PER_W,), jnp.int32),
        pltpu.VMEM((2, CHUNK, D_MODEL), jnp.float32),
        pltpu.SemaphoreType.DMA,
        pltpu.SemaphoreType.DMA,
        pltpu.SemaphoreType.DMA,
        pltpu.SemaphoreType.DMA,
    ],
)
def _sc_gather(comb_hbm, x_hbm, out_hbm, xv, idxv, rowsv,
               gs0, gs1, ws0, ws1):
    wid = lax.axis_index("s") * 2 + lax.axis_index("c")
    base = wid * PER_W
    gsem = (gs0, gs1)
    wsem = (ws0, ws1)

    pltpu.sync_copy(x_hbm.at[pl.ds(base, PER_W)], xv)

    def idx_body(k, carry):
        o = k * LANES
        p = o + lax.iota(jnp.int32, LANES)
        l = lax.rem(p, jnp.int32(L))
        idxv[pl.ds(o, LANES)] = l * NUM_COLORS + xv[pl.ds(o, LANES)]
        return carry

    lax.fori_loop(0, PER_W // LANES, idx_body, 0)

    def gfire(i, b):
        pltpu.async_copy(
            comb_hbm.at[idxv.at[pl.ds(i * CHUNK, CHUNK)]], rowsv.at[b],
            gsem[b])

    def gwait(i, b):
        pltpu.make_async_copy(
            comb_hbm.at[idxv.at[pl.ds(i * CHUNK, CHUNK)]], rowsv.at[b],
            gsem[b]).wait()

    def wfire(i, b):
        pltpu.async_copy(
            rowsv.at[b], out_hbm.at[pl.ds(base + i * CHUNK, CHUNK)], wsem[b])

    def wwait(i, b):
        pltpu.make_async_copy(
            rowsv.at[b], out_hbm.at[pl.ds(base + i * CHUNK, CHUNK)],
            wsem[b]).wait()

    def ring_body(k, carry):
        i0 = k * 2
        i1 = i0 + 1

        @pl.when(k > 0)
        def _():
            gwait(i0 - 1, 1)
            wfire(i0 - 1, 1)
            wwait(i0 - 2, 0)

        gfire(i0, 0)
        gwait(i0, 0)
        wfire(i0, 0)

        @pl.when(k > 0)
        def _():
            wwait(i1 - 2, 1)

        gfire(i1, 1)
        return carry

    lax.fori_loop(0, NCHUNK // 2, ring_body, 0)

    gwait(NCHUNK - 1, 1)
    wfire(NCHUNK - 1, 1)
    wwait(NCHUNK - 2, 0)
    wwait(NCHUNK - 1, 1)


def kernel(x, tok_weight, pos_weight):
    comb = _build_comb(tok_weight, pos_weight)
    comb_flat = comb.reshape(MAX_LEN * NUM_COLORS, D_MODEL)
    x_flat = x.reshape(-1).astype(jnp.int32)
    out = _sc_gather(comb_flat, x_flat)
    return out.reshape(B, L, D_MODEL)


# speedup vs baseline: 5.8869x; 1.6667x over previous
import functools

import jax
import jax.numpy as jnp
from jax import lax
from jax.experimental import pallas as pl
from jax.experimental.pallas import tpu as pltpu
from jax.experimental.pallas import tpu_sc as plsc

NUM_COLORS = 10
D_MODEL = 128
MAX_LEN = 900
B = 1024
L = 900

TOKENS = B * L
NUM_WORKERS = 32
PER_W = TOKENS // NUM_WORKERS
ROWS_W = B // NUM_WORKERS
PAIRS_W = ROWS_W // 2
LANES = 16
LROW_PAD = 912
XBUF = 1824


def _comb_body(tok_ref, pos_ref, out_ref):
    out_ref[...] = pos_ref[...][:, None, :] + tok_ref[...][None, :, :]


def _build_comb(tok, pos):
    return pl.pallas_call(
        _comb_body,
        out_shape=jax.ShapeDtypeStruct((MAX_LEN, NUM_COLORS, D_MODEL),
                                       jnp.float32),
    )(tok, pos)


_sc_mesh = plsc.VectorSubcoreMesh(core_axis_name="c", subcore_axis_name="s")


@functools.partial(
    pl.kernel,
    out_type=jax.ShapeDtypeStruct((B, L, D_MODEL), jnp.float32),
    mesh=_sc_mesh,
    scratch_types=[
        pltpu.VMEM((XBUF,), jnp.int32),
        pltpu.VMEM((1920,), jnp.int32),
        pltpu.VMEM((L, D_MODEL), jnp.float32),
        pltpu.SemaphoreType.DMA,
        pltpu.SemaphoreType.DMA,
    ],
)
def _sc_gather(comb_hbm, x_hbm, out_hbm, xv, idxv, rowsv, gsem, wsem):
    s = lax.axis_index("s")
    wid = s * 2 + lax.axis_index("c")
    base = wid * PER_W
    row0 = wid * ROWS_W

    def wwait():
        pltpu.make_async_copy(rowsv, out_hbm.at[0], wsem).wait()

    def pair_body(pr, carry):
        pp = lax.rem(pr + s, jnp.int32(PAIRS_W))

        pltpu.sync_copy(x_hbm.at[pl.ds(base + pp * (2 * L), 2 * L)],
                        xv.at[pl.ds(0, 2 * L)])

        lane = lax.iota(jnp.int32, LANES)
        rot4 = lax.rem(lane + 4, jnp.int32(LANES))

        def _lane_take(v, idx):
            return lax.gather(
                v, idx[:, None],
                dimension_numbers=lax.GatherDimensionNumbers(
                    offset_dims=(), collapsed_slice_dims=(0,),
                    start_index_map=(0,)),
                slice_sizes=(1,),
                mode=lax.GatherScatterMode.PROMISE_IN_BOUNDS)

        def idx0_body(k, c):
            o = k * LANES
            l = o + lane
            idxv[pl.ds(o, LANES)] = l * NUM_COLORS + xv[pl.ds(o, LANES)]
            return c

        lax.fori_loop(0, LROW_PAD // LANES, idx0_body, 0)

        def idx1_body(k, c):
            o = k * LANES
            l = o + lane
            va = xv[pl.ds(896 + o, LANES)]
            vb = xv[pl.ds(912 + o, LANES)]
            xval = jnp.where(lane < 12, _lane_take(va, rot4),
                             _lane_take(vb, rot4))
            idxv[pl.ds(LROW_PAD + o, LANES)] = l * NUM_COLORS + xval
            return c

        lax.fori_loop(0, LROW_PAD // LANES, idx1_body, 0)

        for sub in range(2):
            if sub == 0:
                @pl.when(pr > 0)
                def _():
                    wwait()
            else:
                wwait()

            ib = sub * LROW_PAD
            for j in range(7):
                pltpu.async_copy(
                    comb_hbm.at[idxv.at[pl.ds(ib + j * 120, 120)]],
                    rowsv.at[pl.ds(j * 120, 120)], gsem)
            pltpu.async_copy(
                comb_hbm.at[idxv.at[pl.ds(ib + 840, 60)]],
                rowsv.at[pl.ds(840, 60)], gsem)
            for j in range(7):
                pltpu.make_async_copy(
                    comb_hbm.at[idxv.at[pl.ds(j * 120, 120)]],
                    rowsv.at[pl.ds(j * 120, 120)], gsem).wait()
            pltpu.make_async_copy(
                comb_hbm.at[idxv.at[pl.ds(840, 60)]],
                rowsv.at[pl.ds(840, 60)], gsem).wait()

            pltpu.async_copy(rowsv, out_hbm.at[row0 + pp * 2 + sub], wsem)

        return carry

    lax.fori_loop(0, PAIRS_W, pair_body, 0)
    wwait()


def kernel(x, tok_weight, pos_weight):
    comb = _build_comb(tok_weight, pos_weight)
    comb_flat = comb.reshape(MAX_LEN * NUM_COLORS, D_MODEL)
    x_flat = x.reshape(-1).astype(jnp.int32)
    return _sc_gather(comb_flat, x_flat)
